# baseline (speedup 1.0000x reference)
import jax
import jax.numpy as jnp
from jax import lax
from jax.experimental import pallas as pl
from jax.experimental.pallas import tpu as pltpu
from jax.experimental.pallas import tpu_sc as plsc

_NC = 2
_NS = 16
_CH = 128
_NBUF = 2
_LOOK = 1
_NH = 2

_SELU_ALPHA = 1.6732632423543772
_SELU_LAM = 1.0507009873554805


def _sc_segsum(with_deg, n_pad, d, cpb):
    rps = n_pad // _NS
    cph = cpb // _NH

    out_type = [jax.ShapeDtypeStruct((_NC, n_pad, d), jnp.float32)]
    scratch = [
        pltpu.VMEM((cph, _CH), jnp.int32),
        pltpu.VMEM((cph, _CH), jnp.int32),
        pltpu.VMEM_SHARED((n_pad, d), jnp.float32),
    ]
    scratch += [pltpu.VMEM((_CH, d), jnp.float32) for _ in range(_NBUF)]
    scratch += [pltpu.SemaphoreType.DMA for _ in range(2 * _NBUF)]
    if with_deg:
        out_type.append(jax.ShapeDtypeStruct((_NC, n_pad), jnp.float32))
        scratch += [
            pltpu.VMEM((_CH,), jnp.float32),
            pltpu.VMEM_SHARED((n_pad,), jnp.float32),
        ]
        scratch += [pltpu.SemaphoreType.DMA for _ in range(_NBUF)]

    mesh = plsc.VectorSubcoreMesh(core_axis_name="c", subcore_axis_name="s")

    def body(*refs):
        if with_deg:
            h_hbm, src_hbm, dst_hbm, z2_hbm, z1_hbm = refs[:5]
            agg_out, deg_out = refs[5:7]
            refs = refs[7:]
        else:
            h_hbm, src_hbm, dst_hbm, z2_hbm = refs[:4]
            (agg_out,) = refs[4:5]
            refs = refs[5:]
        src_blk, dst_blk, acc = refs[:3]
        rows = refs[3:3 + _NBUF]
        sems = refs[3 + _NBUF:3 + 2 * _NBUF]
        ssems = refs[3 + 2 * _NBUF:3 + 3 * _NBUF]
        if with_deg:
            ones_v, dacc = refs[3 + 3 * _NBUF:3 + 3 * _NBUF + 2]
            dsems = refs[3 + 3 * _NBUF + 2:]

        cid = lax.axis_index("c")
        sid = lax.axis_index("s")
        base = pl.multiple_of(sid * rps, 8)

        pltpu.sync_copy(z2_hbm.at[pl.ds(base, rps)], acc.at[pl.ds(base, rps)])
        if with_deg:
            pltpu.sync_copy(z1_hbm.at[pl.ds(base, rps)], dacc.at[pl.ds(base, rps)])
            for i in range(_CH // 16):
                ones_v[pl.ds(i * 16, 16)] = jnp.ones((16,), jnp.float32)
        plsc.subcore_barrier()

        wid = sid * _NC + cid

        def issue(j, b):
            pltpu.async_copy(h_hbm.at[src_blk.at[j]], rows[b], sems[b])

        def work(j, b):
            pltpu.make_async_copy(h_hbm.at[src_blk.at[j]], rows[b],
                                  sems[b]).wait()
            pltpu.async_copy(rows[b], acc.at[dst_blk.at[j]], ssems[b],
                             add=True)
            if with_deg:
                pltpu.async_copy(ones_v, dacc.at[dst_blk.at[j]], dsems[b],
                                 add=True)

        def retire(j2, bn):
            pltpu.make_async_copy(rows[bn], acc.at[dst_blk.at[j2]],
                                  ssems[bn]).wait()
            if with_deg:
                pltpu.make_async_copy(ones_v, dacc.at[dst_blk.at[j2]],
                                      dsems[bn]).wait()

        ng = cph // _NBUF
        for hh in range(_NH):
            cbase = pl.multiple_of(wid * cpb + hh * cph, 8)
            pltpu.sync_copy(src_hbm.at[pl.ds(cbase, cph)], src_blk)
            pltpu.sync_copy(dst_hbm.at[pl.ds(cbase, cph)], dst_blk)

            for b in range(_LOOK):
                issue(b, b)
            for b in range(_NBUF):
                work(b, b)
                bn = (b + _LOOK) % _NBUF
                if b >= _LOOK:
                    retire(b - _LOOK, bn)
                issue(b + _LOOK, bn)

            def group(g, carry):
                for b in range(_NBUF):
                    j = g * _NBUF + b
                    work(j, b)
                    bn = (b + _LOOK) % _NBUF
                    retire(j - _LOOK, bn)
                    issue(j + _LOOK, bn)
                return carry

            lax.fori_loop(1, ng - 1, group, 0)

            for b in range(_NBUF):
                j = (ng - 1) * _NBUF + b
                work(j, b)
                bn = (b + _LOOK) % _NBUF
                retire(j - _LOOK, bn)
                if b < _NBUF - _LOOK:
                    issue(j + _LOOK, bn)
            for b in range(_NBUF - _LOOK, _NBUF):
                retire((ng - 1) * _NBUF + b, b)

        plsc.subcore_barrier()
        pltpu.sync_copy(acc.at[pl.ds(base, rps)],
                        agg_out.at[cid, pl.ds(base, rps)])
        if with_deg:
            pltpu.sync_copy(dacc.at[pl.ds(base, rps)],
                            deg_out.at[cid, pl.ds(base, rps)])

    return pl.kernel(body, out_type=tuple(out_type), mesh=mesh,
                     scratch_types=tuple(scratch))


def _tc_layer(h, parts, deg_parts, w_self, w_neigh, b, act, block_rows):
    n_pad, d = h.shape
    hdim = w_self.shape[1]
    grid = n_pad // block_rows

    def body(h_ref, p_ref, dg_ref, ws_ref, wn_ref, b_ref, o_ref):
        hb = h_ref[...]
        agg = p_ref[0] + p_ref[1]
        deg = dg_ref[0] + dg_ref[1]
        mean = agg * (1.0 / jnp.maximum(deg, 1.0))
        y = (jnp.dot(hb, ws_ref[...], preferred_element_type=jnp.float32)
             + jnp.dot(mean, wn_ref[...], preferred_element_type=jnp.float32)
             + b_ref[...])
        if act == "selu":
            o_ref[...] = jnp.where(
                y > 0.0, _SELU_LAM * y,
                (_SELU_LAM * _SELU_ALPHA) * (jnp.exp(y) - 1.0))
        else:
            m = jnp.max(y, axis=1, keepdims=True)
            ey = jnp.exp(y - m)
            o_ref[...] = ey / jnp.sum(ey, axis=1, keepdims=True)

    return pl.pallas_call(
        body,
        grid=(grid,),
        in_specs=[
            pl.BlockSpec((block_rows, d), lambda i: (i, 0)),
            pl.BlockSpec((_NC, block_rows, d), lambda i: (0, i, 0)),
            pl.BlockSpec((_NC, block_rows, 1), lambda i: (0, i, 0)),
            pl.BlockSpec((d, hdim), lambda i: (0, 0)),
            pl.BlockSpec((d, hdim), lambda i: (0, 0)),
            pl.BlockSpec((1, hdim), lambda i: (0, 0)),
        ],
        out_specs=pl.BlockSpec((block_rows, hdim), lambda i: (i, 0)),
        out_shape=jax.ShapeDtypeStruct((n_pad, hdim), jnp.float32),
    )(h, parts, deg_parts, w_self, w_neigh, b)


def kernel(x, edge_index, W_self0, W_neigh0, b0, W_self1, W_neigh1, b1,
           W_self2, W_neigh2, b2):
    n, d = x.shape
    e = edge_index.shape[1]
    n_pad = -(-n // 1280) * 1280

    nw = _NC * _NS
    grain = nw * _CH * _NH * 8
    e_pad = -(-e // grain) * grain
    cpb = e_pad // (nw * _CH)

    src = edge_index[0]
    dst = edge_index[1]
    pad_i = jnp.arange(e_pad - e, dtype=jnp.int32)
    src_p = jnp.concatenate([src, pad_i % n])
    dst_p = jnp.concatenate([dst, n + pad_i % (n_pad - n)])
    src2d = src_p.reshape(e_pad // _CH, _CH)
    dst2d = dst_p.reshape(e_pad // _CH, _CH)

    xp = jnp.zeros((n_pad, d), jnp.float32).at[:n].set(x)
    z2 = jnp.zeros((n_pad, d), jnp.float32)
    z1 = jnp.zeros((n_pad,), jnp.float32)

    seg_deg = _sc_segsum(True, n_pad, d, cpb)
    seg = _sc_segsum(False, n_pad, d, cpb)

    agg0, deg = seg_deg(xp, src2d, dst2d, z2, z1)
    degr = deg.reshape(_NC, n_pad, 1)
    b0r = b0.reshape(1, -1)
    b1r = b1.reshape(1, -1)
    b2r = b2.reshape(1, -1)

    h1 = _tc_layer(xp, agg0, degr, W_self0, W_neigh0, b0r, "selu", 640)
    (agg1,) = seg(h1, src2d, dst2d, z2)
    h2 = _tc_layer(h1, agg1, degr, W_self1, W_neigh1, b1r, "selu", 640)
    (agg2,) = seg(h2, src2d, dst2d, z2)
    out = _tc_layer(h2, agg2, degr, W_self2, W_neigh2, b2r, "softmax", 640)
    return out[:n]


# speedup vs baseline: 1.1481x; 1.1481x over previous
import jax
import jax.numpy as jnp
from jax import lax
from jax.experimental import pallas as pl
from jax.experimental.pallas import tpu as pltpu
from jax.experimental.pallas import tpu_sc as plsc

_NC = 2
_NS = 16
_CH = 128
_NBUF = 2
_NH = 2

_SELU_ALPHA = 1.6732632423543772
_SELU_LAM = 1.0507009873554805


def _sc_segsum(with_deg, n_pad, d, cpb):
    rps = n_pad // _NS
    cph = cpb // _NH

    out_type = [jax.ShapeDtypeStruct((_NC, n_pad, d), jnp.float32)]
    scratch = [
        pltpu.VMEM((cph, _CH), jnp.int32),
        pltpu.VMEM((cph, _CH), jnp.int32),
        pltpu.VMEM_SHARED((n_pad, d), jnp.float32),
    ]
    scratch += [pltpu.VMEM((_CH, d), jnp.float32) for _ in range(_NBUF)]
    scratch += [pltpu.SemaphoreType.DMA for _ in range(_NBUF)]
    if with_deg:
        out_type.append(jax.ShapeDtypeStruct((_NC, n_pad), jnp.float32))
        scratch += [
            pltpu.VMEM((_CH,), jnp.float32),
            pltpu.VMEM_SHARED((n_pad,), jnp.float32),
        ]

    mesh = plsc.VectorSubcoreMesh(core_axis_name="c", subcore_axis_name="s")

    def body(*refs):
        if with_deg:
            h_hbm, src_hbm, dst_hbm, z2_hbm, z1_hbm = refs[:5]
            agg_out, deg_out = refs[5:7]
            refs = refs[7:]
        else:
            h_hbm, src_hbm, dst_hbm, z2_hbm = refs[:4]
            (agg_out,) = refs[4:5]
            refs = refs[5:]
        src_blk, dst_blk, acc = refs[:3]
        rows = refs[3:3 + _NBUF]
        sems = refs[3 + _NBUF:3 + 2 * _NBUF]
        if with_deg:
            ones_v, dacc = refs[3 + 2 * _NBUF:]

        cid = lax.axis_index("c")
        sid = lax.axis_index("s")
        base = pl.multiple_of(sid * rps, 8)

        pltpu.sync_copy(z2_hbm.at[pl.ds(base, rps)], acc.at[pl.ds(base, rps)])
        if with_deg:
            pltpu.sync_copy(z1_hbm.at[pl.ds(base, rps)], dacc.at[pl.ds(base, rps)])
            for i in range(_CH // 16):
                ones_v[pl.ds(i * 16, 16)] = jnp.ones((16,), jnp.float32)
        plsc.subcore_barrier()

        wid = sid * _NC + cid

        def scat(j, b):
            pltpu.make_async_copy(h_hbm.at[src_blk.at[j]], rows[b],
                                  sems[b]).wait()
            pltpu.sync_copy(rows[b], acc.at[dst_blk.at[j]], add=True)
            if with_deg:
                pltpu.sync_copy(ones_v, dacc.at[dst_blk.at[j]], add=True)

        for hh in range(_NH):
            cbase = pl.multiple_of(wid * cpb + hh * cph, 8)
            pltpu.sync_copy(src_hbm.at[pl.ds(cbase, cph)], src_blk)
            pltpu.sync_copy(dst_hbm.at[pl.ds(cbase, cph)], dst_blk)

            for b in range(_NBUF):
                pltpu.async_copy(h_hbm.at[src_blk.at[b]], rows[b], sems[b])

            def group(g, carry):
                for b in range(_NBUF):
                    j = g * _NBUF + b
                    scat(j, b)
                    pltpu.async_copy(h_hbm.at[src_blk.at[j + _NBUF]],
                                     rows[b], sems[b])
                return carry

            lax.fori_loop(0, cph // _NBUF - 1, group, 0)
            for b in range(_NBUF):
                scat((cph // _NBUF - 1) * _NBUF + b, b)

        plsc.subcore_barrier()
        pltpu.sync_copy(acc.at[pl.ds(base, rps)],
                        agg_out.at[cid, pl.ds(base, rps)])
        if with_deg:
            pltpu.sync_copy(dacc.at[pl.ds(base, rps)],
                            deg_out.at[cid, pl.ds(base, rps)])

    return pl.kernel(body, out_type=tuple(out_type), mesh=mesh,
                     scratch_types=tuple(scratch))


def _tc_layer(h, parts, deg_parts, w_self, w_neigh, b, act, block_rows):
    n_pad, d = h.shape
    hdim = w_self.shape[1]
    grid = n_pad // block_rows

    def body(h_ref, p_ref, dg_ref, ws_ref, wn_ref, b_ref, o_ref):
        hb = h_ref[...]
        agg = p_ref[0] + p_ref[1]
        deg = dg_ref[0] + dg_ref[1]
        mean = agg * (1.0 / jnp.maximum(deg, 1.0))
        y = (jnp.dot(hb, ws_ref[...], preferred_element_type=jnp.float32)
             + jnp.dot(mean, wn_ref[...], preferred_element_type=jnp.float32)
             + b_ref[...])
        if act == "selu":
            o_ref[...] = jnp.where(
                y > 0.0, _SELU_LAM * y,
                (_SELU_LAM * _SELU_ALPHA) * (jnp.exp(y) - 1.0))
        else:
            m = jnp.max(y, axis=1, keepdims=True)
            ey = jnp.exp(y - m)
            o_ref[...] = ey / jnp.sum(ey, axis=1, keepdims=True)

    return pl.pallas_call(
        body,
        grid=(grid,),
        in_specs=[
            pl.BlockSpec((block_rows, d), lambda i: (i, 0)),
            pl.BlockSpec((_NC, block_rows, d), lambda i: (0, i, 0)),
            pl.BlockSpec((_NC, block_rows, 1), lambda i: (0, i, 0)),
            pl.BlockSpec((d, hdim), lambda i: (0, 0)),
            pl.BlockSpec((d, hdim), lambda i: (0, 0)),
            pl.BlockSpec((1, hdim), lambda i: (0, 0)),
        ],
        out_specs=pl.BlockSpec((block_rows, hdim), lambda i: (i, 0)),
        out_shape=jax.ShapeDtypeStruct((n_pad, hdim), jnp.float32),
    )(h, parts, deg_parts, w_self, w_neigh, b)


def kernel(x, edge_index, W_self0, W_neigh0, b0, W_self1, W_neigh1, b1,
           W_self2, W_neigh2, b2):
    n, d = x.shape
    e = edge_index.shape[1]
    n_pad = -(-n // 1280) * 1280

    nw = _NC * _NS
    grain = nw * _CH * _NH * 8
    e_pad = -(-e // grain) * grain
    cpb = e_pad // (nw * _CH)

    src = edge_index[0]
    dst = edge_index[1]
    pad_i = jnp.arange(e_pad - e, dtype=jnp.int32)
    src_p = jnp.concatenate([src, pad_i % n])
    dst_p = jnp.concatenate([dst, n + pad_i % (n_pad - n)])
    src2d = src_p.reshape(e_pad // _CH, _CH)
    dst2d = dst_p.reshape(e_pad // _CH, _CH)

    xp = jnp.zeros((n_pad, d), jnp.float32).at[:n].set(x)
    z2 = jnp.zeros((n_pad, d), jnp.float32)
    z1 = jnp.zeros((n_pad,), jnp.float32)

    seg_deg = _sc_segsum(True, n_pad, d, cpb)
    seg = _sc_segsum(False, n_pad, d, cpb)

    agg0, deg = seg_deg(xp, src2d, dst2d, z2, z1)
    degr = deg.reshape(_NC, n_pad, 1)
    b0r = b0.reshape(1, -1)
    b1r = b1.reshape(1, -1)
    b2r = b2.reshape(1, -1)

    h1 = _tc_layer(xp, agg0, degr, W_self0, W_neigh0, b0r, "selu", 640)
    (agg1,) = seg(h1, src2d, dst2d, z2)
    h2 = _tc_layer(h1, agg1, degr, W_self1, W_neigh1, b1r, "selu", 640)
    (agg2,) = seg(h2, src2d, dst2d, z2)
    out = _tc_layer(h2, agg2, degr, W_self2, W_neigh2, b2r, "softmax", 640)
    return out[:n]
